# baseline (speedup 1.0000x reference)
import functools

import jax
import jax.numpy as jnp
from jax.experimental import pallas as pl
from jax.experimental.pallas import tpu as pltpu


def _rpn_level_kernel(x_ref, wt_ref, hw_ref, cb_ref, hb_ref, lg_ref, bb_ref,
                      xbf_ref, x3_ref, *, W, SEG, S):
    cb = cb_ref[...]
    hb = hb_ref[...]
    lane = jax.lax.broadcasted_iota(jnp.int32, (256, SEG), 1)
    lane_w = lane % W

    def cast_chunk(c, carry):
        xbf_ref[:, pl.ds(c * SEG, SEG)] = (
            x_ref[:, pl.ds(c * SEG, SEG)].astype(jnp.bfloat16))
        return carry

    jax.lax.fori_loop(0, S, cast_chunk, 0)

    def row_view(j0, o, first, last):
        if o == 0:
            return xbf_ref[:, pl.ds(j0, SEG)]
        if o < 0 and first:
            chunk = xbf_ref[:, pl.ds(j0, SEG)]
            cp = pltpu.roll(chunk, -o, axis=1)
            return jnp.where(lane < -o, jnp.bfloat16(0), cp)
        if o > 0 and last:
            chunk = xbf_ref[:, pl.ds(j0, SEG)]
            cp = pltpu.roll(chunk, SEG - o, axis=1)
            return jnp.where(lane >= SEG - o, jnp.bfloat16(0), cp)
        base, r = (o // 128) * 128, o % 128
        if r == 0:
            return xbf_ref[:, pl.ds(j0 + base, SEG)]
        chunk = xbf_ref[:, pl.ds(j0 + base, SEG + 128)]
        return pltpu.roll(chunk, SEG + 128 - r, axis=1)[:, :SEG]

    def build(buf, j0, first, last):
        for dy in range(3):
            cp = row_view(j0, (dy - 1) * W, first, last)
            x3_ref[buf, dy * 256:(dy + 1) * 256, :] = cp

    build(0, 0, True, S == 1)

    def seg_step(s, carry):
        p = jax.lax.rem(s, 2)

        @pl.when((s + 1 >= 1) & (s + 1 < S - 1))
        def _():
            build(1 - p, (s + 1) * SEG, False, False)

        if S > 1:
            @pl.when(s + 1 == S - 1)
            def _():
                build(1 - p, (S - 1) * SEG, False, True)

        r3 = x3_ref[p]
        v0 = jax.lax.dot_general(
            wt_ref[0], r3, (((1,), (0,)), ((), ())),
            preferred_element_type=jnp.float32)
        v1 = jax.lax.dot_general(
            wt_ref[1], r3, (((1,), (0,)), ((), ())),
            preferred_element_type=jnp.float32)
        v2 = jax.lax.dot_general(
            wt_ref[2], r3, (((1,), (0,)), ((), ())),
            preferred_element_type=jnp.float32)
        vm = jnp.where(lane_w == 0, 0.0, pltpu.roll(v0, 1, axis=1))
        vp = jnp.where(lane_w == W - 1, 0.0, pltpu.roll(v2, SEG - 1, axis=1))
        t = jnp.maximum(v1 + vm + vp + cb, 0.0).astype(jnp.bfloat16)
        o = jax.lax.dot_general(
            hw_ref[...], t, (((1,), (0,)), ((), ())),
            preferred_element_type=jnp.float32) + hb
        lg_ref[:, pl.ds(s * SEG, SEG)] = o[0:3]
        bb_ref[:, pl.ds(s * SEG, SEG)] = o[8:20]
        return carry

    jax.lax.fori_loop(0, S, seg_step, 0)


def _run_level(x, wt, hw, cb, hb, SEG):
    N, C, H, W = x.shape
    Lr = H * W
    assert Lr % SEG == 0 and SEG % W == 0
    assert Lr // SEG <= 2 or SEG >= 256
    S = Lr // SEG
    xf = x.reshape(N, C, Lr)
    lg, bb = pl.pallas_call(
        functools.partial(_rpn_level_kernel, W=W, SEG=SEG, S=S),
        grid=(N,),
        in_specs=[
            pl.BlockSpec((None, C, Lr), lambda b: (b, 0, 0)),
            pl.BlockSpec((3, C, 3 * C), lambda b: (0, 0, 0)),
            pl.BlockSpec((24, C), lambda b: (0, 0)),
            pl.BlockSpec((C, 1), lambda b: (0, 0)),
            pl.BlockSpec((24, 1), lambda b: (0, 0)),
        ],
        out_specs=[
            pl.BlockSpec((None, 3, Lr), lambda b: (b, 0, 0)),
            pl.BlockSpec((None, 12, Lr), lambda b: (b, 0, 0)),
        ],
        out_shape=[
            jax.ShapeDtypeStruct((N, 3, Lr), jnp.float32),
            jax.ShapeDtypeStruct((N, 12, Lr), jnp.float32),
        ],
        scratch_shapes=[
            pltpu.VMEM((C, Lr), jnp.bfloat16),
            pltpu.VMEM((2, 3 * C, SEG), jnp.bfloat16),
        ],
        compiler_params=pltpu.CompilerParams(
            dimension_semantics=("parallel",)),
    )(xf, wt, hw, cb, hb)
    return lg.reshape(N, 3, H, W), bb.reshape(N, 12, H, W)


_LEVEL_SEG = (1024, 2048, 1024, 256, 64)


def kernel(feature0, feature1, feature2, feature3, feature4,
           conv_w, conv_b, cls_w, cls_b, bbox_w, bbox_b):
    wt = conv_w.transpose(3, 0, 2, 1).reshape(3, 256, 768).astype(jnp.bfloat16)
    z5 = jnp.zeros((5, 256), cls_w.dtype)
    z4 = jnp.zeros((4, 256), cls_w.dtype)
    hw = jnp.concatenate(
        [cls_w[:, :, 0, 0], z5, bbox_w[:, :, 0, 0], z4]).astype(jnp.bfloat16)
    cb = conv_b.reshape(256, 1)
    hb = jnp.concatenate(
        [cls_b, jnp.zeros((5,), cls_b.dtype), bbox_b,
         jnp.zeros((4,), cls_b.dtype)]).reshape(24, 1)
    logits, bbox = [], []
    for f, seg in zip((feature0, feature1, feature2, feature3, feature4),
                      _LEVEL_SEG):
        lo, bb = _run_level(f, wt, hw, cb, hb, seg)
        logits.append(lo)
        bbox.append(bb)
    return tuple(logits) + tuple(bbox)
